# baseline (speedup 1.0000x reference)
import functools

import jax
import jax.numpy as jnp
from jax import lax
from jax.experimental import pallas as pl
from jax.experimental.pallas import tpu as pltpu
from jax.experimental.pallas import tpu_sc as plsc

N_NODES = 10000
N_EDGES = 160000
D_IN = 256
D_OUT = 256
DQ = D_OUT // 4
LANES = 16
NSUB = 16
K = 80
EDGES_PER_TILE = N_EDGES // NSUB
NCHUNK = EDGES_PER_TILE // K
NPAD = 10240
ROWS_PER_TILE = NPAD // NSUB
ZR = 128
NJQ = DQ // LANES
NB = 5


def _mm_body(x_ref, w_ref, f0_ref, f1_ref, f2_ref, f3_ref):
    f = jnp.dot(x_ref[...], w_ref[...], preferred_element_type=jnp.float32)
    f0_ref[...] = f[:, 0 * DQ:1 * DQ]
    f1_ref[...] = f[:, 1 * DQ:2 * DQ]
    f2_ref[...] = f[:, 2 * DQ:3 * DQ]
    f3_ref[...] = f[:, 3 * DQ:4 * DQ]


def _matmul(x, weight):
    n = x.shape[0]
    br = 1000
    return pl.pallas_call(
        _mm_body,
        grid=(n // br,),
        in_specs=[
            pl.BlockSpec((br, D_IN), lambda i: (i, 0)),
            pl.BlockSpec((D_IN, D_OUT), lambda i: (0, 0)),
        ],
        out_specs=[pl.BlockSpec((br, DQ), lambda i: (i, 0))] * 4,
        out_shape=[jax.ShapeDtypeStruct((n, DQ), jnp.float32)] * 4,
    )(x, weight)


def _sc_body(f00, f01, f10, f11, src3, dst3, ew3, wb,
             s00, s01, s10, s11, cnt,
             src_v, dst_v, ew_v, rows_v, ones_v, zbuf_v, zcnt_v, wbv,
             acc, cnt_acc, sem_g0, sem_g1, sem_g2, sem_g3, sem_g4,
             sem_s0, sem_s1, sem_s2, sem_s3, sem_s4, sem_c):
    c = lax.axis_index("c")
    s = lax.axis_index("s")
    base = s * ROWS_PER_TILE
    rows_sl = pl.ds(base, ROWS_PER_TILE)
    sem_g = (sem_g0, sem_g1, sem_g2, sem_g3, sem_g4)
    sem_s = (sem_s0, sem_s1, sem_s2, sem_s3, sem_s4)

    def _zrow(i, carry):
        for j in range(NJQ):
            zbuf_v[i, pl.ds(j * LANES, LANES)] = jnp.zeros((LANES,), jnp.float32)
        zcnt_v[i, :] = jnp.zeros((LANES,), jnp.float32)
        return carry

    lax.fori_loop(0, ZR, _zrow, 0)

    def _orow(i, carry):
        ones_v[i, :] = jnp.ones((LANES,), jnp.float32)
        return carry

    lax.fori_loop(0, K, _orow, 0)

    pltpu.sync_copy(src3.at[s], src_v)
    pltpu.sync_copy(dst3.at[s], dst_v)
    pltpu.sync_copy(ew3.at[s], ew_v)

    @pl.when(c == 0)
    def _():
        pltpu.sync_copy(wb.at[0], wbv)

    @pl.when(c == 1)
    def _():
        pltpu.sync_copy(wb.at[1], wbv)

    def _pass(feat_ref, sum_ref, p, do_cnt):
        for t in range(ROWS_PER_TILE // ZR):
            pltpu.sync_copy(zbuf_v, acc.at[pl.ds(base + t * ZR, ZR)])
        if do_cnt:
            for t in range(ROWS_PER_TILE // ZR):
                pltpu.sync_copy(zcnt_v, cnt_acc.at[pl.ds(base + t * ZR, ZR)])
        plsc.subcore_barrier()

        wej = [wbv[p, 0, pl.ds(j * LANES, LANES)] for j in range(NJQ)]
        bej = [wbv[p, 1, pl.ds(j * LANES, LANES)] for j in range(NJQ)]

        def _fire_g(ii, b):
            pltpu.async_copy(feat_ref.at[src_v.at[ii]], rows_v.at[b], sem_g[b])

        def _wait_g(ii, b):
            pltpu.make_async_copy(
                feat_ref.at[src_v.at[ii]], rows_v.at[b], sem_g[b]).wait()

        def _fire_s(ii, b):
            pltpu.async_copy(rows_v.at[b], acc.at[dst_v.at[ii]], sem_s[b],
                             add=True)

        def _wait_s(ii, b):
            pltpu.make_async_copy(
                rows_v.at[b], acc.at[dst_v.at[ii]], sem_s[b]).wait()

        def _step(ii, b):
            _wait_g(ii, b)
            bn = (b + NB - 1) % NB

            @pl.when(ii >= 1)
            def _():
                _wait_s(ii - 1, bn)

            @pl.when(ii + NB - 1 < NCHUNK)
            def _():
                _fire_g(ii + NB - 1, bn)

            @plsc.parallel_loop(0, K // LANES, unroll=1)
            def _grp(g):
                ew16 = ew_v[ii, pl.ds(g * LANES, LANES)]
                for i2 in range(LANES):
                    ewk = ew16[i2]
                    k = g * LANES + i2
                    for j in range(NJQ):
                        t = ewk * wej[j] + bej[j]
                        sl = pl.ds(j * LANES, LANES)
                        rows_v[b, k, sl] = rows_v[b, k, sl] * t

            _fire_s(ii, b)
            if do_cnt:
                pltpu.async_copy(ones_v, cnt_acc.at[dst_v.at[ii]], sem_c,
                                 add=True)

                @pl.when(ii >= 2)
                def _():
                    pltpu.make_async_copy(
                        ones_v, cnt_acc.at[dst_v.at[ii]], sem_c).wait()

        for b0 in range(NB - 1):
            _fire_g(jnp.int32(b0), b0)

        def _round(ir, carry):
            for r in range(NB):
                _step(ir * NB + r, r)
            return carry

        lax.fori_loop(0, NCHUNK // NB, _round, 0)
        _wait_s(jnp.int32(NCHUNK - 1), (NCHUNK - 1) % NB)
        if do_cnt:
            for ii in (NCHUNK - 2, NCHUNK - 1):
                pltpu.make_async_copy(
                    ones_v, cnt_acc.at[dst_v.at[ii]], sem_c).wait()
        plsc.subcore_barrier()

        pltpu.sync_copy(acc.at[rows_sl], sum_ref.at[rows_sl])
        if do_cnt:
            pltpu.sync_copy(cnt_acc.at[rows_sl], cnt.at[rows_sl])
        plsc.subcore_barrier()

    @pl.when(c == 0)
    def _():
        _pass(f00, s00, 0, True)
        _pass(f01, s01, 1, False)

    @pl.when(c == 1)
    def _():
        _pass(f10, s10, 0, False)
        _pass(f11, s11, 1, False)


_SC_SCRATCH = [
    pltpu.VMEM((NCHUNK, K), jnp.int32),
    pltpu.VMEM((NCHUNK, K), jnp.int32),
    pltpu.VMEM((NCHUNK, K), jnp.float32),
    pltpu.VMEM((NB, K, DQ), jnp.float32),
    pltpu.VMEM((K, LANES), jnp.float32),
    pltpu.VMEM((ZR, DQ), jnp.float32),
    pltpu.VMEM((ZR, LANES), jnp.float32),
    pltpu.VMEM((2, 2, DQ), jnp.float32),
    pltpu.VMEM_SHARED((NPAD, DQ), jnp.float32),
    pltpu.VMEM_SHARED((NPAD, LANES), jnp.float32),
] + [pltpu.SemaphoreType.DMA] * (2 * NB + 1) + [
]

_sc_call = functools.partial(
    pl.kernel,
    out_type=(
        jax.ShapeDtypeStruct((NPAD, DQ), jnp.float32),
        jax.ShapeDtypeStruct((NPAD, DQ), jnp.float32),
        jax.ShapeDtypeStruct((NPAD, DQ), jnp.float32),
        jax.ShapeDtypeStruct((NPAD, DQ), jnp.float32),
        jax.ShapeDtypeStruct((NPAD, LANES), jnp.float32),
    ),
    mesh=plsc.VectorSubcoreMesh(core_axis_name="c", subcore_axis_name="s",
                                num_cores=2, num_subcores=NSUB),
    scratch_types=_SC_SCRATCH,
    compiler_params=pltpu.CompilerParams(use_tc_tiling_on_sc=False),
)(_sc_body)


def _fin_body(s0_ref, s1_ref, s2_ref, s3_ref, cnt_ref, out_ref):
    inv = 1.0 / jnp.maximum(cnt_ref[:, 0:1], 1.0)
    out_ref[:, 0 * DQ:1 * DQ] = jnp.maximum(s0_ref[...] * inv, 0.0)
    out_ref[:, 1 * DQ:2 * DQ] = jnp.maximum(s1_ref[...] * inv, 0.0)
    out_ref[:, 2 * DQ:3 * DQ] = jnp.maximum(s2_ref[...] * inv, 0.0)
    out_ref[:, 3 * DQ:4 * DQ] = jnp.maximum(s3_ref[...] * inv, 0.0)


def _finalize(s00, s01, s10, s11, cnt):
    n = N_NODES
    br = 1000
    return pl.pallas_call(
        _fin_body,
        grid=(n // br,),
        in_specs=[pl.BlockSpec((br, DQ), lambda i: (i, 0))] * 4
        + [pl.BlockSpec((br, LANES), lambda i: (i, 0))],
        out_specs=pl.BlockSpec((br, D_OUT), lambda i: (i, 0)),
        out_shape=jax.ShapeDtypeStruct((n, D_OUT), jnp.float32),
    )(s00, s01, s10, s11, cnt)


def kernel(x, edge_index, edge_w, weight, We, be):
    src = edge_index[0].astype(jnp.int32).reshape(NSUB, NCHUNK, K)
    dst = edge_index[1].astype(jnp.int32).reshape(NSUB, NCHUNK, K)
    ew = edge_w.reshape(NSUB, NCHUNK, K)
    f00, f01, f10, f11 = _matmul(x, weight)
    wq = We[:, 0].reshape(2, 2, DQ)
    bq = be.reshape(2, 2, DQ)
    wb = jnp.stack([wq, bq], axis=2)
    s00, s01, s10, s11, cnt = _sc_call(
        f00, f01, f10, f11, src, dst, ew, wb)
    return _finalize(s00, s01, s10, s11, cnt)


# speedup vs baseline: 1.1643x; 1.1643x over previous
import functools

import jax
import jax.numpy as jnp
from jax import lax
from jax.experimental import pallas as pl
from jax.experimental.pallas import tpu as pltpu
from jax.experimental.pallas import tpu_sc as plsc

N_NODES = 10000
N_EDGES = 160000
D_IN = 256
D_OUT = 256
DQ = D_OUT // 4
LANES = 16
NSUB = 16
K = 80
EDGES_PER_TILE = N_EDGES // NSUB
NCHUNK = EDGES_PER_TILE // K
NPAD = 10240
ROWS_PER_TILE = NPAD // NSUB
ZR = 128
NJQ = DQ // LANES
NB = 5


def _mm_body(x_ref, w_ref, f0_ref, f1_ref, f2_ref, f3_ref):
    f = jnp.dot(x_ref[...], w_ref[...], preferred_element_type=jnp.float32)
    f0_ref[...] = f[:, 0 * DQ:1 * DQ]
    f1_ref[...] = f[:, 1 * DQ:2 * DQ]
    f2_ref[...] = f[:, 2 * DQ:3 * DQ]
    f3_ref[...] = f[:, 3 * DQ:4 * DQ]


def _matmul(x, weight):
    n = x.shape[0]
    br = 1000
    return pl.pallas_call(
        _mm_body,
        grid=(n // br,),
        in_specs=[
            pl.BlockSpec((br, D_IN), lambda i: (i, 0)),
            pl.BlockSpec((D_IN, D_OUT), lambda i: (0, 0)),
        ],
        out_specs=[pl.BlockSpec((br, DQ), lambda i: (i, 0))] * 4,
        out_shape=[jax.ShapeDtypeStruct((n, DQ), jnp.float32)] * 4,
    )(x, weight)


def _sc_body(f00, f01, f10, f11, src3, dst3, ew3, wb,
             s00, s01, s10, s11, cnt,
             src_v, dst_v, ew_v, rows_v, ones_v, zbuf_v, zcnt_v, wbv,
             acc, cnt_acc, *sems):
    c = lax.axis_index("c")
    s = lax.axis_index("s")
    base = s * ROWS_PER_TILE
    rows_sl = pl.ds(base, ROWS_PER_TILE)
    sem_g = sems[:NB]
    sem_s = sems[NB:2 * NB]
    sem_c = sems[2 * NB]
    sem_g2 = sems[2 * NB + 1:]

    def _zrow(i, carry):
        for j in range(NJQ):
            zbuf_v[i, pl.ds(j * LANES, LANES)] = jnp.zeros((LANES,), jnp.float32)
        zcnt_v[i, :] = jnp.zeros((LANES,), jnp.float32)
        return carry

    lax.fori_loop(0, ZR, _zrow, 0)

    def _orow(i, carry):
        ones_v[i, :] = jnp.ones((LANES,), jnp.float32)
        return carry

    lax.fori_loop(0, K, _orow, 0)

    pltpu.sync_copy(src3.at[s], src_v)
    pltpu.sync_copy(dst3.at[s], dst_v)
    pltpu.sync_copy(ew3.at[s], ew_v)

    @pl.when(c == 0)
    def _():
        pltpu.sync_copy(wb.at[0], wbv)

    @pl.when(c == 1)
    def _():
        pltpu.sync_copy(wb.at[1], wbv)

    def _pass(feat_ref, sum_ref, p, do_cnt):
        for t in range(ROWS_PER_TILE // ZR):
            pltpu.sync_copy(zbuf_v, acc.at[pl.ds(base + t * ZR, ZR)])
        if do_cnt:
            for t in range(ROWS_PER_TILE // ZR):
                pltpu.sync_copy(zcnt_v, cnt_acc.at[pl.ds(base + t * ZR, ZR)])
        plsc.subcore_barrier()

        wej = [wbv[p, 0, pl.ds(j * LANES, LANES)] for j in range(NJQ)]
        bej = [wbv[p, 1, pl.ds(j * LANES, LANES)] for j in range(NJQ)]

        KH = K // 2

        def _fire_g(ii, b):
            pltpu.async_copy(feat_ref.at[src_v.at[ii, pl.ds(0, KH)]],
                             rows_v.at[b, pl.ds(0, KH)], sem_g[b])
            pltpu.async_copy(feat_ref.at[src_v.at[ii, pl.ds(KH, KH)]],
                             rows_v.at[b, pl.ds(KH, KH)], sem_g2[b])

        def _wait_g(ii, b):
            pltpu.make_async_copy(
                feat_ref.at[src_v.at[ii, pl.ds(0, KH)]],
                rows_v.at[b, pl.ds(0, KH)], sem_g[b]).wait()
            pltpu.make_async_copy(
                feat_ref.at[src_v.at[ii, pl.ds(KH, KH)]],
                rows_v.at[b, pl.ds(KH, KH)], sem_g2[b]).wait()

        def _fire_s(ii, b):
            pltpu.async_copy(rows_v.at[b], acc.at[dst_v.at[ii]], sem_s[b],
                             add=True)

        def _wait_s(ii, b):
            pltpu.make_async_copy(
                rows_v.at[b], acc.at[dst_v.at[ii]], sem_s[b]).wait()

        def _step(ii, b):
            _wait_g(ii, b)
            bn = (b + NB - 2) % NB

            @pl.when(ii >= 2)
            def _():
                _wait_s(ii - 2, bn)

            @pl.when(ii + NB - 2 < NCHUNK)
            def _():
                _fire_g(ii + NB - 2, bn)

            @plsc.parallel_loop(0, K // LANES, unroll=1)
            def _grp(g):
                ew16 = ew_v[ii, pl.ds(g * LANES, LANES)]
                for i2 in range(LANES):
                    ewk = ew16[i2]
                    k = g * LANES + i2
                    for j in range(NJQ):
                        t = ewk * wej[j] + bej[j]
                        sl = pl.ds(j * LANES, LANES)
                        rows_v[b, k, sl] = rows_v[b, k, sl] * t

            _fire_s(ii, b)
            if do_cnt:
                pltpu.async_copy(ones_v, cnt_acc.at[dst_v.at[ii]], sem_c,
                                 add=True)

                @pl.when(ii >= 2)
                def _():
                    pltpu.make_async_copy(
                        ones_v, cnt_acc.at[dst_v.at[ii]], sem_c).wait()

        for b0 in range(NB - 2):
            _fire_g(jnp.int32(b0), b0)

        def _round(ir, carry):
            for r in range(NB):
                _step(ir * NB + r, r)
            return carry

        lax.fori_loop(0, NCHUNK // NB, _round, 0)
        for ii in (NCHUNK - 2, NCHUNK - 1):
            _wait_s(jnp.int32(ii), ii % NB)
        if do_cnt:
            for ii in (NCHUNK - 2, NCHUNK - 1):
                pltpu.make_async_copy(
                    ones_v, cnt_acc.at[dst_v.at[ii]], sem_c).wait()
        plsc.subcore_barrier()

        pltpu.sync_copy(acc.at[rows_sl], sum_ref.at[rows_sl])
        if do_cnt:
            pltpu.sync_copy(cnt_acc.at[rows_sl], cnt.at[rows_sl])
        plsc.subcore_barrier()

    @pl.when(c == 0)
    def _():
        _pass(f00, s00, 0, True)
        _pass(f01, s01, 1, False)

    @pl.when(c == 1)
    def _():
        _pass(f10, s10, 0, False)
        _pass(f11, s11, 1, False)


_SC_SCRATCH = [
    pltpu.VMEM((NCHUNK, K), jnp.int32),
    pltpu.VMEM((NCHUNK, K), jnp.int32),
    pltpu.VMEM((NCHUNK, K), jnp.float32),
    pltpu.VMEM((NB, K, DQ), jnp.float32),
    pltpu.VMEM((K, LANES), jnp.float32),
    pltpu.VMEM((ZR, DQ), jnp.float32),
    pltpu.VMEM((ZR, LANES), jnp.float32),
    pltpu.VMEM((2, 2, DQ), jnp.float32),
    pltpu.VMEM_SHARED((NPAD, DQ), jnp.float32),
    pltpu.VMEM_SHARED((NPAD, LANES), jnp.float32),
] + [pltpu.SemaphoreType.DMA] * (3 * NB + 1) + [
]

_sc_call = functools.partial(
    pl.kernel,
    out_type=(
        jax.ShapeDtypeStruct((NPAD, DQ), jnp.float32),
        jax.ShapeDtypeStruct((NPAD, DQ), jnp.float32),
        jax.ShapeDtypeStruct((NPAD, DQ), jnp.float32),
        jax.ShapeDtypeStruct((NPAD, DQ), jnp.float32),
        jax.ShapeDtypeStruct((NPAD, LANES), jnp.float32),
    ),
    mesh=plsc.VectorSubcoreMesh(core_axis_name="c", subcore_axis_name="s",
                                num_cores=2, num_subcores=NSUB),
    scratch_types=_SC_SCRATCH,
    compiler_params=pltpu.CompilerParams(use_tc_tiling_on_sc=False),
)(_sc_body)


def _fin_body(s0_ref, s1_ref, s2_ref, s3_ref, cnt_ref, out_ref):
    inv = 1.0 / jnp.maximum(cnt_ref[:, 0:1], 1.0)
    out_ref[:, 0 * DQ:1 * DQ] = jnp.maximum(s0_ref[...] * inv, 0.0)
    out_ref[:, 1 * DQ:2 * DQ] = jnp.maximum(s1_ref[...] * inv, 0.0)
    out_ref[:, 2 * DQ:3 * DQ] = jnp.maximum(s2_ref[...] * inv, 0.0)
    out_ref[:, 3 * DQ:4 * DQ] = jnp.maximum(s3_ref[...] * inv, 0.0)


def _finalize(s00, s01, s10, s11, cnt):
    n = N_NODES
    br = 1000
    return pl.pallas_call(
        _fin_body,
        grid=(n // br,),
        in_specs=[pl.BlockSpec((br, DQ), lambda i: (i, 0))] * 4
        + [pl.BlockSpec((br, LANES), lambda i: (i, 0))],
        out_specs=pl.BlockSpec((br, D_OUT), lambda i: (i, 0)),
        out_shape=jax.ShapeDtypeStruct((n, D_OUT), jnp.float32),
    )(s00, s01, s10, s11, cnt)


def kernel(x, edge_index, edge_w, weight, We, be):
    src = edge_index[0].astype(jnp.int32).reshape(NSUB, NCHUNK, K)
    dst = edge_index[1].astype(jnp.int32).reshape(NSUB, NCHUNK, K)
    ew = edge_w.reshape(NSUB, NCHUNK, K)
    f00, f01, f10, f11 = _matmul(x, weight)
    wq = We[:, 0].reshape(2, 2, DQ)
    bq = be.reshape(2, 2, DQ)
    wb = jnp.stack([wq, bq], axis=2)
    s00, s01, s10, s11, cnt = _sc_call(
        f00, f01, f10, f11, src, dst, ew, wb)
    return _finalize(s00, s01, s10, s11, cnt)
